# baseline (speedup 1.0000x reference)
import jax
import jax.numpy as jnp
from jax import lax
from jax.experimental import pallas as pl
from jax.experimental.pallas import tpu as pltpu
from jax.experimental.pallas import tpu_sc as plsc

_N_COMP, _N_NODES, _N_T = 16, 325, 12
_W = _N_COMP * _N_NODES * _N_T
_B = 64
_NC, _NS, _L = 2, 16, 16
_NG = _B // _L
_NN = 168
_N0_H1 = _N_NODES - _NN
_JN = _NN * _N_T
_HR, _DR = 24, 7
_HP, _DP = 32, 8
_HW = _JN * _HP + 1024
_DW = _JN * _DP + 2304
_CHN = 8
_NPAIR = 10


def _body(hm_hbm, dm_hbm, ht_hbm, dt_hbm, out_hbm,
          hm_v, dm_v, hstage, dstage, sem_t, s0, s1):
    wid = lax.axis_index("s") * _NC + lax.axis_index("c")
    comp = wid // 2
    half = wid % 2
    n0 = half * _N0_H1
    j0 = (comp * _N_NODES + n0) * _N_T

    offh = jnp.minimum((j0 * _HP) // 1024 * 1024, _W * _HP - _HW)
    offh = pl.multiple_of(offh, 1024)
    shifth = j0 * _HP - offh
    offd = jnp.minimum((j0 * _DP) // 1024 * 1024, _W * _DP - _DW)
    offd = pl.multiple_of(offd, 1024)
    shiftd = j0 * _DP - offd

    pltpu.sync_copy(hm_hbm, hm_v)
    pltpu.sync_copy(dm_hbm, dm_v)
    cp_h = pltpu.async_copy(ht_hbm.at[pl.ds(offh, _HW)], hstage, sem_t)
    cp_d = pltpu.async_copy(dt_hbm.at[pl.ds(offd, _DW)], dstage, sem_t)
    cp_h.wait()
    cp_d.wait()

    def _scoped(ob0, ob1):
        _compute(out_hbm, hstage, dstage, hm_v, dm_v,
                 comp, n0, shifth, shiftd, (ob0, ob1), (s0, s1))

    pl.run_scoped(_scoped,
                  pltpu.VMEM((_CHN, _N_T, _B), jnp.float32),
                  pltpu.VMEM((_CHN, _N_T, _B), jnp.float32))


def _lane_shuffle(vec, idx):
    dnums = lax.GatherDimensionNumbers(
        offset_dims=(), collapsed_slice_dims=(0,), start_index_map=(0,))
    return lax.gather(vec, idx[:, None], dnums, (1,),
                      mode=lax.GatherScatterMode.PROMISE_IN_BOUNDS)


def _compute(out_hbm, hstage, dstage, hm_v, dm_v,
             comp, n0, shifth, shiftd, obufs, sems):
    hlo, hhi, hmask, dlo, dhi = [], [], [], [], []
    for g in range(_NG):
        hm = hm_v[g]
        dm = dm_v[g]
        hlo.append(hm)
        hhi.append(jnp.maximum(hm - _L, 0))
        hmask.append(hm < _L)
        dlo.append(dm)
        dhi.append(dm + _DP)

    def _chunk(ci, j, drain):
        ob = obufs[j]
        if drain is True:
            pltpu.make_async_copy(ob, out_hbm.at[comp, pl.ds(0, _CHN)],
                                  sems[j]).wait()
        elif drain is not None:
            @pl.when(drain)
            def _drain():
                pltpu.make_async_copy(ob, out_hbm.at[comp, pl.ds(0, _CHN)],
                                      sems[j]).wait()
        base = ci * _CHN

        @plsc.parallel_loop(0, _N_T // 2, unroll=2)
        def _pairloop(p):
            t = p * 2
            for nj in range(_CHN):
                jl = (base + nj) * _N_T + t
                hb = jl * _HP + shifth
                db = jl * _DP + shiftd
                dv = dstage[pl.ds(db, _L)]
                for half_t in range(2):
                    hv0 = hstage[pl.ds(hb + half_t * _HP, _L)]
                    hv1 = hstage[pl.ds(hb + half_t * _HP + _L, _L)]
                    for g in range(_NG):
                        hs = jnp.where(hmask[g],
                                       _lane_shuffle(hv0, hlo[g]),
                                       _lane_shuffle(hv1, hhi[g]))
                        ds_ = _lane_shuffle(
                            dv, dlo[g] if half_t == 0 else dhi[g])
                        ob[nj, t + half_t, pl.ds(g * _L, _L)] = hs + ds_

        pltpu.async_copy(ob, out_hbm.at[comp, pl.ds(n0 + base, _CHN)],
                         sems[j])

    def _pair(i, carry):
        base2 = i * 2
        _chunk(base2, 0, i > 0)
        _chunk(base2 + 1, 1, i > 0)
        return carry

    lax.fori_loop(0, _NPAIR, _pair, 0)
    _chunk(2 * _NPAIR, 0, True)
    pltpu.make_async_copy(obufs[0], out_hbm.at[comp, pl.ds(0, _CHN)],
                          sems[0]).wait()
    pltpu.make_async_copy(obufs[1], out_hbm.at[comp, pl.ds(0, _CHN)],
                          sems[1]).wait()


@jax.jit
def _run(H, D, h_ebd, d_ebd):
    hm = H.reshape(_NG, _L).astype(jnp.int32)
    dm = D.reshape(_NG, _L).astype(jnp.int32)
    ht = jnp.pad(h_ebd, ((0, _HP - _HR), (0, 0))).T.reshape(_W * _HP)
    dt = jnp.pad(d_ebd, ((0, _DP - _DR), (0, 0))).T.reshape(_W * _DP)
    mesh = plsc.VectorSubcoreMesh(core_axis_name="c", subcore_axis_name="s")
    out = pl.kernel(
        _body,
        out_type=jax.ShapeDtypeStruct((_N_COMP, _N_NODES, _N_T, _B),
                                      jnp.float32),
        mesh=mesh,
        compiler_params=pltpu.CompilerParams(needs_layout_passes=False,
                                             disable_bounds_checks=True),
        scratch_types=[
            pltpu.VMEM((_NG, _L), jnp.int32),
            pltpu.VMEM((_NG, _L), jnp.int32),
            pltpu.VMEM((_HW,), jnp.float32),
            pltpu.VMEM((_DW,), jnp.float32),
            pltpu.SemaphoreType.DMA,
            pltpu.SemaphoreType.DMA,
            pltpu.SemaphoreType.DMA,
        ],
    )(hm, dm, ht, dt)
    return jnp.transpose(out, (3, 0, 1, 2))


def kernel(H, D, h_ebd, d_ebd):
    return _run(H, D, h_ebd, d_ebd)


# speedup vs baseline: 1.0970x; 1.0970x over previous
import jax
import jax.numpy as jnp
from jax import lax
from jax.experimental import pallas as pl
from jax.experimental.pallas import tpu as pltpu
from jax.experimental.pallas import tpu_sc as plsc

_N_COMP, _N_NODES, _N_T = 16, 325, 12
_W = _N_COMP * _N_NODES * _N_T
_B = 64
_NC, _NS, _L = 2, 16, 16
_NG = _B // _L
_NN = 168
_N0_H1 = _N_NODES - _NN
_JN = _NN * _N_T
_HR, _DR = 24, 7
_P = 48
_HDW = 97280
_CHN = 8
_NPAIR = 10


def _body(hm_hbm, dm_hbm, hdt_hbm, out_hbm,
          hm_v, dm_v, stage, sem_t, s0, s1):
    wid = lax.axis_index("s") * _NC + lax.axis_index("c")
    comp = wid // 2
    half = wid % 2
    n0 = half * _N0_H1
    j0 = (comp * _N_NODES + n0) * _N_T

    off = jnp.minimum((j0 * _P) // 1024 * 1024, _W * _P - _HDW)
    off = pl.multiple_of(off, 1024)
    shift = j0 * _P - off

    pltpu.sync_copy(hm_hbm, hm_v)
    pltpu.sync_copy(dm_hbm, dm_v)
    pltpu.async_copy(hdt_hbm.at[pl.ds(off, _HDW)], stage, sem_t).wait()

    def _scoped(ob0, ob1):
        _compute(out_hbm, stage, hm_v, dm_v,
                 comp, n0, shift, (ob0, ob1), (s0, s1))

    pl.run_scoped(_scoped,
                  pltpu.VMEM((_CHN, _N_T, _B), jnp.float32),
                  pltpu.VMEM((_CHN, _N_T, _B), jnp.float32))


def _lane_shuffle(vec, idx):
    dnums = lax.GatherDimensionNumbers(
        offset_dims=(), collapsed_slice_dims=(0,), start_index_map=(0,))
    return lax.gather(vec, idx[:, None], dnums, (1,),
                      mode=lax.GatherScatterMode.PROMISE_IN_BOUNDS)


def _compute(out_hbm, stage, hm_v, dm_v, comp, n0, shift, obufs, sems):
    hlo, hhi, hmask, dsh = [], [], [], []
    for g in range(_NG):
        hm = hm_v[g]
        dm = dm_v[g]
        hlo.append(hm)
        hhi.append(jnp.maximum(hm - _L, 0))
        hmask.append(hm < _L)
        dsh.append(dm + _HR - _L)

    def _chunk(ci, j, drain):
        ob = obufs[j]
        if drain is True:
            pltpu.make_async_copy(ob, out_hbm.at[comp, pl.ds(0, _CHN)],
                                  sems[j]).wait()
        elif drain is not None:
            @pl.when(drain)
            def _drain():
                pltpu.make_async_copy(ob, out_hbm.at[comp, pl.ds(0, _CHN)],
                                      sems[j]).wait()
        base = ci * _CHN

        @plsc.parallel_loop(0, _N_T, unroll=2)
        def _tloop(t):
            for nj in range(_CHN):
                jl = (base + nj) * _N_T + t
                hb = jl * _P + shift
                hv0 = stage[pl.ds(hb, _L)]
                hv1 = stage[pl.ds(hb + _L, _L)]
                for g in range(_NG):
                    hs = jnp.where(hmask[g],
                                   _lane_shuffle(hv0, hlo[g]),
                                   _lane_shuffle(hv1, hhi[g]))
                    ds_ = _lane_shuffle(hv1, dsh[g])
                    ob[nj, t, pl.ds(g * _L, _L)] = hs + ds_

        pltpu.async_copy(ob, out_hbm.at[comp, pl.ds(n0 + base, _CHN)],
                         sems[j])

    def _pair(i, carry):
        base2 = i * 2
        _chunk(base2, 0, i > 0)
        _chunk(base2 + 1, 1, i > 0)
        return carry

    lax.fori_loop(0, _NPAIR, _pair, 0)
    _chunk(2 * _NPAIR, 0, True)
    pltpu.make_async_copy(obufs[0], out_hbm.at[comp, pl.ds(0, _CHN)],
                          sems[0]).wait()
    pltpu.make_async_copy(obufs[1], out_hbm.at[comp, pl.ds(0, _CHN)],
                          sems[1]).wait()


@jax.jit
def _run(H, D, h_ebd, d_ebd):
    hm = H.reshape(_NG, _L).astype(jnp.int32)
    dm = D.reshape(_NG, _L).astype(jnp.int32)
    hd = jnp.concatenate([h_ebd, d_ebd], axis=0)
    hdt = jnp.pad(hd, ((0, _P - _HR - _DR), (0, 0))).T.reshape(_W * _P)
    mesh = plsc.VectorSubcoreMesh(core_axis_name="c", subcore_axis_name="s")
    out = pl.kernel(
        _body,
        out_type=jax.ShapeDtypeStruct((_N_COMP, _N_NODES, _N_T, _B),
                                      jnp.float32),
        mesh=mesh,
        compiler_params=pltpu.CompilerParams(needs_layout_passes=False,
                                             disable_bounds_checks=True),
        scratch_types=[
            pltpu.VMEM((_NG, _L), jnp.int32),
            pltpu.VMEM((_NG, _L), jnp.int32),
            pltpu.VMEM((_HDW,), jnp.float32),
            pltpu.SemaphoreType.DMA,
            pltpu.SemaphoreType.DMA,
            pltpu.SemaphoreType.DMA,
        ],
    )(hm, dm, hdt)
    return jnp.transpose(out, (3, 0, 1, 2))


def kernel(H, D, h_ebd, d_ebd):
    return _run(H, D, h_ebd, d_ebd)


# speedup vs baseline: 1.0977x; 1.0006x over previous
import jax
import jax.numpy as jnp
from jax import lax
from jax.experimental import pallas as pl
from jax.experimental.pallas import tpu as pltpu
from jax.experimental.pallas import tpu_sc as plsc

_N_COMP, _N_NODES, _N_T = 16, 325, 12
_W = _N_COMP * _N_NODES * _N_T
_B = 64
_NC, _NS, _L = 2, 16, 16
_NG = _B // _L
_NN = 168
_N0_H1 = _N_NODES - _NN
_JN = _NN * _N_T
_HR, _DR = 24, 7
_P = 48
_HDW = 98304
_CHN = 8
_NPAIR = 10


def _body(hm_hbm, dm_hbm, hdt_hbm, out_hbm,
          hm_v, dm_v, stage, sem_t, s0, s1):
    wid = lax.axis_index("s") * _NC + lax.axis_index("c")
    comp = wid // 2
    half = wid % 2
    n0 = half * _N0_H1
    j0 = (comp * _N_NODES + n0) * _N_T

    off = jnp.minimum((j0 * _P) // 1024 * 1024, _W * _P - _HDW)
    off = pl.multiple_of(off, 1024)
    shift = j0 * _P - off

    pltpu.sync_copy(hm_hbm, hm_v)
    pltpu.sync_copy(dm_hbm, dm_v)
    pltpu.async_copy(hdt_hbm.at[pl.ds(off, _HDW)], stage, sem_t).wait()

    def _scoped(ob0, ob1):
        _compute(out_hbm, stage, hm_v, dm_v,
                 comp, n0, shift, (ob0, ob1), (s0, s1))

    pl.run_scoped(_scoped,
                  pltpu.VMEM((_CHN, _N_T, _B), jnp.float32),
                  pltpu.VMEM((_CHN, _N_T, _B), jnp.float32))


def _lane_shuffle(vec, idx):
    dnums = lax.GatherDimensionNumbers(
        offset_dims=(), collapsed_slice_dims=(0,), start_index_map=(0,))
    return lax.gather(vec, idx[:, None], dnums, (1,),
                      mode=lax.GatherScatterMode.PROMISE_IN_BOUNDS)


def _compute(out_hbm, stage, hm_v, dm_v, comp, n0, shift, obufs, sems):
    hlo, hhi, hmask, dsh = [], [], [], []
    for g in range(_NG):
        hm = hm_v[g]
        dm = dm_v[g]
        hlo.append(hm)
        hhi.append(jnp.maximum(hm - _L, 0))
        hmask.append(hm < _L)
        dsh.append(dm + _HR - _L)

    def _chunk(ci, j, drain):
        ob = obufs[j]
        if drain is True:
            pltpu.make_async_copy(ob, out_hbm.at[comp, pl.ds(0, _CHN)],
                                  sems[j]).wait()
        elif drain is not None:
            @pl.when(drain)
            def _drain():
                pltpu.make_async_copy(ob, out_hbm.at[comp, pl.ds(0, _CHN)],
                                      sems[j]).wait()
        base = ci * _CHN

        @plsc.parallel_loop(0, _N_T, unroll=2)
        def _tloop(t):
            for nj in range(_CHN):
                jl = (base + nj) * _N_T + t
                hb = jl * _P + shift
                hv0 = stage[pl.ds(hb, _L)]
                hv1 = stage[pl.ds(hb + _L, _L)]
                for g in range(_NG):
                    hs = jnp.where(hmask[g],
                                   _lane_shuffle(hv0, hlo[g]),
                                   _lane_shuffle(hv1, hhi[g]))
                    ds_ = _lane_shuffle(hv1, dsh[g])
                    ob[nj, t, pl.ds(g * _L, _L)] = hs + ds_

        pltpu.async_copy(ob, out_hbm.at[comp, pl.ds(n0 + base, _CHN)],
                         sems[j])

    def _pair(i, carry):
        base2 = i * 2
        _chunk(base2, 0, i > 0)
        _chunk(base2 + 1, 1, i > 0)
        return carry

    lax.fori_loop(0, _NPAIR, _pair, 0)
    _chunk(2 * _NPAIR, 0, True)
    pltpu.make_async_copy(obufs[0], out_hbm.at[comp, pl.ds(0, _CHN)],
                          sems[0]).wait()
    pltpu.make_async_copy(obufs[1], out_hbm.at[comp, pl.ds(0, _CHN)],
                          sems[1]).wait()


@jax.jit
def _run(H, D, h_ebd, d_ebd):
    hm = H.reshape(_NG, _L).astype(jnp.int32)
    dm = D.reshape(_NG, _L).astype(jnp.int32)
    hd = jnp.concatenate([h_ebd, d_ebd], axis=0)
    hdt = jnp.pad(hd, ((0, _P - _HR - _DR), (0, 0))).T.reshape(_W * _P)
    mesh = plsc.VectorSubcoreMesh(core_axis_name="c", subcore_axis_name="s")
    out = pl.kernel(
        _body,
        out_type=jax.ShapeDtypeStruct((_N_COMP, _N_NODES, _N_T, _B),
                                      jnp.float32),
        mesh=mesh,
        compiler_params=pltpu.CompilerParams(needs_layout_passes=False,
                                             disable_bounds_checks=True),
        scratch_types=[
            pltpu.VMEM((_NG, _L), jnp.int32),
            pltpu.VMEM((_NG, _L), jnp.int32),
            pltpu.VMEM((_HDW,), jnp.float32),
            pltpu.SemaphoreType.DMA,
            pltpu.SemaphoreType.DMA,
            pltpu.SemaphoreType.DMA,
        ],
    )(hm, dm, hdt)
    return jnp.transpose(out, (3, 0, 1, 2))


def kernel(H, D, h_ebd, d_ebd):
    return _run(H, D, h_ebd, d_ebd)
